# baseline (speedup 1.0000x reference)
import jax
import jax.numpy as jnp
from jax import lax
from jax.experimental import pallas as pl

POOL_SIZE = 64
LENGTH = 16
EMBED_DIM = 1024
TOP_K = 8
BATCH = 128
TAU = 5.0
NEG_INF = -3.0e38


def _tc_body(cls_ref, pk_ref, prompt_ref, bp_ref, sim_ref, keys_ref, idx_ref,
             pool_ref):
    cls = cls_ref[...]
    pk = pk_ref[...]
    eps = 1e-12

    xn = cls * lax.rsqrt(jnp.maximum(jnp.sum(cls * cls, axis=1, keepdims=True), eps))
    pn = pk * lax.rsqrt(jnp.maximum(jnp.sum(pk * pk, axis=1, keepdims=True), eps))

    sim = lax.dot_general(xn, pn, (((1,), (1,)), ((), ())),
                          preferred_element_type=jnp.float32)
    sim_ref[...] = sim

    z = (sim - jnp.max(sim, axis=1, keepdims=True)) * (1.0 / TAU)
    e = jnp.exp(z)
    w = e / jnp.sum(e, axis=1, keepdims=True)
    bp_ref[...] = jnp.dot(w, prompt_ref[...], preferred_element_type=jnp.float32)

    col = lax.broadcasted_iota(jnp.int32, (BATCH, POOL_SIZE), 1)
    kcol = lax.broadcasted_iota(jnp.int32, (BATCH, TOP_K), 1)
    vals = sim
    selected = jnp.zeros((BATCH, POOL_SIZE), dtype=jnp.bool_)
    idx_acc = jnp.zeros((BATCH, TOP_K), dtype=jnp.int32)
    for k in range(TOP_K):
        m = jnp.max(vals, axis=1, keepdims=True)
        cand = jnp.where(vals == m, col, POOL_SIZE)
        sel = jnp.min(cand, axis=1, keepdims=True)
        hit = col == sel
        vals = jnp.where(hit, NEG_INF, vals)
        selected = jnp.logical_or(selected, hit)
        idx_acc = jnp.where(kcol == k, sel, idx_acc)
        keys_ref[:, k * EMBED_DIM:(k + 1) * EMBED_DIM] = jnp.dot(
            hit.astype(jnp.float32), pn, preferred_element_type=jnp.float32)
    idx_ref[...] = idx_acc
    pool_ref[...] = jnp.sum(selected.astype(jnp.float32), axis=0,
                            keepdims=True)


def kernel(x_embed, cls_features, prompt, prompt_key, cur_task, train_mode):
    del x_embed, cur_task, train_mode
    prompt_flat = prompt.reshape(POOL_SIZE, LENGTH * EMBED_DIM)
    bp, sim, keys, idx, pool = pl.pallas_call(
        _tc_body,
        out_shape=(
            jax.ShapeDtypeStruct((BATCH, LENGTH * EMBED_DIM), jnp.float32),
            jax.ShapeDtypeStruct((BATCH, POOL_SIZE), jnp.float32),
            jax.ShapeDtypeStruct((BATCH, TOP_K * EMBED_DIM), jnp.float32),
            jax.ShapeDtypeStruct((BATCH, TOP_K), jnp.int32),
            jax.ShapeDtypeStruct((1, POOL_SIZE), jnp.float32),
        ),
    )(cls_features, prompt_key, prompt_flat)
    return (bp.reshape(BATCH, LENGTH, EMBED_DIM), sim,
            keys.reshape(BATCH, TOP_K, EMBED_DIM), idx, pool.reshape(POOL_SIZE))
